# baseline (speedup 1.0000x reference)
import functools

import jax
import jax.numpy as jnp
from jax import lax
from jax.experimental import pallas as pl
from jax.experimental.pallas import tpu as pltpu
from jax.experimental.pallas import tpu_sc as plsc

N = 10000
E = 160000
N_USERS = 5000
IN_DIM, H1, H2 = 256, 128, 64

NC, NS = 2, 16
EPC = E // NC
EPT = EPC // NS
B = 64
NB = EPT // B
TB = EPT - NB * B
ZR = 104
RPT = 624


def _make_spmm(D):
    mesh = plsc.VectorSubcoreMesh(core_axis_name="c", subcore_axis_name="s")

    @functools.partial(
        pl.kernel,
        out_type=jax.ShapeDtypeStruct((NC, N, D), jnp.float32),
        mesh=mesh,
        scratch_types=[
            pltpu.VMEM((EPT,), jnp.int32),
            pltpu.VMEM((EPT + 16,), jnp.float32),
            pltpu.VMEM((B,), jnp.int32),
            pltpu.VMEM((B,), jnp.int32),
            pltpu.VMEM((B,), jnp.int32),
            pltpu.VMEM((TB,), jnp.int32),
            pltpu.VMEM((B, D), jnp.float32),
            pltpu.VMEM((B, D), jnp.float32),
            pltpu.VMEM((B, D), jnp.float32),
            pltpu.VMEM((TB, D), jnp.float32),
            pltpu.VMEM((ZR, D), jnp.float32),
            pltpu.VMEM_SHARED((N, D), jnp.float32),
            pltpu.SemaphoreType.DMA,
            pltpu.SemaphoreType.DMA,
            pltpu.SemaphoreType.DMA,
            pltpu.SemaphoreType.DMA,
            pltpu.SemaphoreType.DMA,
            pltpu.SemaphoreType.DMA,
            pltpu.SemaphoreType.DMA,
        ],
    )
    def spmm(x_hbm, rows_hbm, cols_hbm, vals_hbm, out_hbm,
             cols_v, vals_v, rows0_v, rows1_v, rows2_v, rowst_v,
             msgs0_v, msgs1_v, msgs2_v, msgst_v, zeros_v, acc_sh,
             sem0a, sem0b, sem1a, sem1b, sem2a, sem2b, semt):
        c = lax.axis_index("c")
        s = lax.axis_index("s")
        wid = c * NS + s
        row0 = s * RPT
        ebase = wid * EPT

        slots = ((rows0_v, msgs0_v, sem0a, sem0b),
                 (rows1_v, msgs1_v, sem1a, sem1b),
                 (rows2_v, msgs2_v, sem2a, sem2b))

        pltpu.sync_copy(cols_hbm.at[pl.ds(ebase, EPT)], cols_v)
        pltpu.sync_copy(vals_hbm.at[pl.ds(ebase, EPT)],
                        vals_v.at[pl.ds(0, EPT)])

        zv = jnp.zeros((16,), jnp.float32)

        def zfill(i, carry):
            for j in range(D // 16):
                zeros_v[i, pl.ds(16 * j, 16)] = zv
            return carry

        lax.fori_loop(0, ZR, zfill, 0)

        def zcopy(k, carry):
            pltpu.sync_copy(zeros_v, acc_sh.at[pl.ds(row0 + k * ZR, ZR)])
            return carry

        lax.fori_loop(0, RPT // ZR, zcopy, 0)

        @pl.when(s == NS - 1)
        def _():
            pltpu.sync_copy(zeros_v.at[pl.ds(0, N - NS * RPT)],
                            acc_sh.at[pl.ds(NS * RPT, N - NS * RPT)])

        plsc.subcore_barrier()

        def scale(msgs_b, vbase, nrows):
            @plsc.parallel_loop(0, nrows, step=1, unroll=4)
            def srow(r):
                v = vals_v[pl.ds(vbase + r, 16)][0]
                for j in range(D // 16):
                    sl = pl.ds(16 * j, 16)
                    msgs_b[r, sl] = msgs_b[r, sl] * v

        def issue_g(j, sl):
            rows_b, msgs_b, sem_g, _ = slots[sl]
            pltpu.async_copy(rows_hbm.at[pl.ds(ebase + j * B, B)],
                             rows_b, sem_g)
            pltpu.async_copy(x_hbm.at[cols_v.at[pl.ds(j * B, B)]],
                             msgs_b, sem_g)

        def wait_g(j, sl):
            rows_b, msgs_b, sem_g, _ = slots[sl]
            pltpu.make_async_copy(rows_hbm.at[pl.ds(ebase + j * B, B)],
                                  rows_b, sem_g).wait()
            pltpu.make_async_copy(x_hbm.at[cols_v.at[pl.ds(j * B, B)]],
                                  msgs_b, sem_g).wait()

        def issue_sc(sl):
            rows_b, msgs_b, _, sem_s = slots[sl]
            pltpu.async_copy(msgs_b, acc_sh.at[rows_b], sem_s, add=True)

        def wait_sc(sl):
            rows_b, msgs_b, _, sem_s = slots[sl]
            pltpu.make_async_copy(msgs_b, acc_sh.at[rows_b], sem_s).wait()

        issue_g(0, 0)
        issue_g(1, 1)
        for j in range(NB):
            sl2 = (j + 2) % 3
            if j + 2 < NB:
                if j >= 1:
                    wait_sc(sl2)
                issue_g(j + 2, sl2)
            elif j + 2 == NB:
                if j >= 1:
                    wait_sc(sl2)
                pltpu.async_copy(rows_hbm.at[pl.ds(ebase + NB * B, TB)],
                                 rowst_v, semt)
                pltpu.async_copy(x_hbm.at[cols_v.at[pl.ds(NB * B, TB)]],
                                 msgst_v, semt)
            sj = j % 3
            wait_g(j, sj)
            scale(msgs_b=slots[sj][1], vbase=j * B, nrows=B)
            issue_sc(sj)
        pltpu.make_async_copy(rows_hbm.at[pl.ds(ebase + NB * B, TB)],
                              rowst_v, semt).wait()
        pltpu.make_async_copy(x_hbm.at[cols_v.at[pl.ds(NB * B, TB)]],
                              msgst_v, semt).wait()
        scale(msgst_v, NB * B, TB)
        pltpu.sync_copy(msgst_v, acc_sh.at[rowst_v], add=True)
        for sl in ((NB - 2) % 3, (NB - 1) % 3):
            wait_sc(sl)

        plsc.subcore_barrier()
        pltpu.sync_copy(acc_sh.at[pl.ds(row0, RPT)],
                        out_hbm.at[c, pl.ds(row0, RPT)])

        @pl.when(s == NS - 1)
        def _():
            pltpu.sync_copy(acc_sh.at[pl.ds(NS * RPT, N - NS * RPT)],
                            out_hbm.at[c, pl.ds(NS * RPT, N - NS * RPT)])

    return spmm


_spmm128 = _make_spmm(128)


def _mm1(x, w):
    def body(x_ref, w_ref, o_ref):
        o_ref[...] = jnp.dot(x_ref[...], w_ref[...],
                             preferred_element_type=jnp.float32)

    return pl.pallas_call(
        body,
        grid=(10,),
        in_specs=[pl.BlockSpec((1000, IN_DIM), lambda i: (i, 0)),
                  pl.BlockSpec((IN_DIM, H1), lambda i: (0, 0))],
        out_specs=pl.BlockSpec((1000, H1), lambda i: (i, 0)),
        out_shape=jax.ShapeDtypeStruct((N, H1), jnp.float32),
    )(x, w)


def _mm2(p0, p1, w):
    def body(p0_ref, p1_ref, w_ref, o_ref):
        h = jnp.maximum(p0_ref[...] + p1_ref[...], 0.0)
        o_ref[...] = jnp.dot(h, w_ref[...], preferred_element_type=jnp.float32)

    return pl.pallas_call(
        body,
        grid=(10,),
        in_specs=[pl.BlockSpec((1000, H1), lambda i: (i, 0)),
                  pl.BlockSpec((1000, H1), lambda i: (i, 0)),
                  pl.BlockSpec((H1, 128), lambda i: (0, 0))],
        out_specs=pl.BlockSpec((1000, 128), lambda i: (i, 0)),
        out_shape=jax.ShapeDtypeStruct((N, 128), jnp.float32),
    )(p0, p1, w)


def _decode(q0, q1):
    BU = 1000
    nu = N_USERS // BU

    def body(u0, u1, i0, i1, o_ref):
        zu = u0[...] + u1[...]
        zi = i0[...] + i1[...]
        acc = lax.dot_general(zu, zi, (((1,), (1,)), ((), ())),
                              preferred_element_type=jnp.float32)
        o_ref[...] = 0.5 * jnp.tanh(acc * 0.5) + 0.5

    return pl.pallas_call(
        body,
        grid=(nu,),
        in_specs=[pl.BlockSpec((BU, 128), lambda i: (i, 0)),
                  pl.BlockSpec((BU, 128), lambda i: (i, 0)),
                  pl.BlockSpec((N_USERS, 128), lambda i: (1, 0)),
                  pl.BlockSpec((N_USERS, 128), lambda i: (1, 0))],
        out_specs=pl.BlockSpec((BU, N_USERS), lambda i: (i, 0)),
        out_shape=jax.ShapeDtypeStruct((N_USERS, N_USERS), jnp.float32),
    )(q0, q1, q0, q1)


def kernel(emb_weight, W1, W2, adj_vals, edge_index):
    rows = edge_index[0]
    cols = edge_index[1]
    vals = adj_vals
    w2p = jnp.concatenate(
        [W2, jnp.zeros((H1, 128 - H2), jnp.float32)], axis=1)
    x1 = _mm1(emb_weight, W1)
    p = _spmm128(x1, rows, cols, vals)
    x2 = _mm2(p[0], p[1], w2p)
    q = _spmm128(x2, rows, cols, vals)
    return _decode(q[0], q[1])


# speedup vs baseline: 1.0647x; 1.0647x over previous
import functools

import jax
import jax.numpy as jnp
from jax import lax
from jax.experimental import pallas as pl
from jax.experimental.pallas import tpu as pltpu
from jax.experimental.pallas import tpu_sc as plsc

N = 10000
E = 160000
N_USERS = 5000
IN_DIM, H1, H2 = 256, 128, 64

NC, NS = 2, 16
EPC = E // NC
EPT = EPC // NS
B = 96
NB = EPT // B
TB = EPT - NB * B
ZR = 104
RPT = 624


def _make_spmm(D, d_scale):
    mesh = plsc.VectorSubcoreMesh(core_axis_name="c", subcore_axis_name="s")

    @functools.partial(
        pl.kernel,
        out_type=jax.ShapeDtypeStruct((NC, N, D), jnp.float32),
        mesh=mesh,
        scratch_types=[
            pltpu.VMEM((EPT,), jnp.int32),
            pltpu.VMEM((EPT + 16,), jnp.float32),
            pltpu.VMEM((B,), jnp.int32),
            pltpu.VMEM((B,), jnp.int32),
            pltpu.VMEM((TB,), jnp.int32),
            pltpu.VMEM((B, D), jnp.float32),
            pltpu.VMEM((B, D), jnp.float32),
            pltpu.VMEM((TB, D), jnp.float32),
            pltpu.VMEM((ZR, D), jnp.float32),
            pltpu.VMEM_SHARED((N, D), jnp.float32),
            pltpu.SemaphoreType.DMA,
            pltpu.SemaphoreType.DMA,
            pltpu.SemaphoreType.DMA,
        ],
    )
    def spmm(x_hbm, rows_hbm, cols_hbm, vals_hbm, out_hbm,
             cols_v, vals_v, rows0_v, rows1_v, rowst_v,
             msgs0_v, msgs1_v, msgst_v, zeros_v, acc_sh, sem0, sem1, sem2):
        c = lax.axis_index("c")
        s = lax.axis_index("s")
        wid = c * NS + s
        row0 = s * RPT
        ebase = wid * EPT

        bufs = ((rows0_v, msgs0_v, sem0), (rows1_v, msgs1_v, sem1))

        pltpu.sync_copy(cols_hbm.at[pl.ds(ebase, EPT)], cols_v)
        pltpu.sync_copy(vals_hbm.at[pl.ds(ebase, EPT)],
                        vals_v.at[pl.ds(0, EPT)])

        zv = jnp.zeros((16,), jnp.float32)

        def zfill(i, carry):
            for j in range(D // 16):
                zeros_v[i, pl.ds(16 * j, 16)] = zv
            return carry

        lax.fori_loop(0, ZR, zfill, 0)

        def zcopy(k, carry):
            pltpu.sync_copy(zeros_v, acc_sh.at[pl.ds(row0 + k * ZR, ZR)])
            return carry

        lax.fori_loop(0, RPT // ZR, zcopy, 0)

        @pl.when(s == NS - 1)
        def _():
            pltpu.sync_copy(zeros_v.at[pl.ds(0, N - NS * RPT)],
                            acc_sh.at[pl.ds(NS * RPT, N - NS * RPT)])

        plsc.subcore_barrier()

        def scale(msgs_b, vbase, nrows):
            @plsc.parallel_loop(0, nrows, step=1, unroll=4)
            def srow(r):
                v = vals_v[pl.ds(vbase + r, 16)][0]
                for j in range(d_scale // 16):
                    sl = pl.ds(16 * j, 16)
                    msgs_b[r, sl] = msgs_b[r, sl] * v

        def issue(i, buf):
            rows_b, msgs_b, sem = bufs[buf]
            pltpu.async_copy(rows_hbm.at[pl.ds(ebase + i * B, B)],
                             rows_b, sem)
            pltpu.async_copy(x_hbm.at[cols_v.at[pl.ds(i * B, B)]],
                             msgs_b, sem)

        def consume(i, buf):
            rows_b, msgs_b, sem = bufs[buf]
            pltpu.make_async_copy(rows_hbm.at[pl.ds(ebase + i * B, B)],
                                  rows_b, sem).wait()
            pltpu.make_async_copy(x_hbm.at[cols_v.at[pl.ds(i * B, B)]],
                                  msgs_b, sem).wait()
            scale(msgs_b, i * B, B)
            pltpu.sync_copy(msgs_b, acc_sh.at[rows_b], add=True)

        def issue_tail():
            pltpu.async_copy(rows_hbm.at[pl.ds(ebase + NB * B, TB)],
                             rowst_v, sem2)
            pltpu.async_copy(x_hbm.at[cols_v.at[pl.ds(NB * B, TB)]],
                             msgst_v, sem2)

        def consume_tail():
            pltpu.make_async_copy(rows_hbm.at[pl.ds(ebase + NB * B, TB)],
                                  rowst_v, sem2).wait()
            pltpu.make_async_copy(x_hbm.at[cols_v.at[pl.ds(NB * B, TB)]],
                                  msgst_v, sem2).wait()
            scale(msgst_v, NB * B, TB)
            pltpu.sync_copy(msgst_v, acc_sh.at[rowst_v], add=True)

        issue(0, 0)

        def pair(k, carry):
            i0 = 2 * k
            issue(i0 + 1, 1)
            consume(i0, 0)
            issue(i0 + 2, 0)
            consume(i0 + 1, 1)
            return carry

        if NB % 2 == 1:
            lax.fori_loop(0, (NB - 1) // 2, pair, 0)
            issue_tail()
            consume(NB - 1, 0)
            consume_tail()
        else:
            lax.fori_loop(0, (NB - 2) // 2, pair, 0)
            issue(NB - 1, 1)
            issue_tail()
            consume(NB - 2, 0)
            consume(NB - 1, 1)
            consume_tail()

        plsc.subcore_barrier()
        pltpu.sync_copy(acc_sh.at[pl.ds(row0, RPT)],
                        out_hbm.at[c, pl.ds(row0, RPT)])

        @pl.when(s == NS - 1)
        def _():
            pltpu.sync_copy(acc_sh.at[pl.ds(NS * RPT, N - NS * RPT)],
                            out_hbm.at[c, pl.ds(NS * RPT, N - NS * RPT)])

    return spmm


_spmm128 = _make_spmm(128, 128)
_spmm128h = _make_spmm(128, H2)


def _mm1(x, w):
    def body(x_ref, w_ref, o_ref):
        o_ref[...] = jnp.dot(x_ref[...], w_ref[...],
                             preferred_element_type=jnp.float32)

    return pl.pallas_call(
        body,
        grid=(10,),
        in_specs=[pl.BlockSpec((1000, IN_DIM), lambda i: (i, 0)),
                  pl.BlockSpec((IN_DIM, H1), lambda i: (0, 0))],
        out_specs=pl.BlockSpec((1000, H1), lambda i: (i, 0)),
        out_shape=jax.ShapeDtypeStruct((N, H1), jnp.float32),
    )(x, w)


def _mm2(p0, p1, w):
    def body(p0_ref, p1_ref, w_ref, o_ref):
        h = jnp.maximum(p0_ref[...] + p1_ref[...], 0.0)
        o_ref[...] = jnp.dot(h, w_ref[...], preferred_element_type=jnp.float32)

    return pl.pallas_call(
        body,
        grid=(10,),
        in_specs=[pl.BlockSpec((1000, H1), lambda i: (i, 0)),
                  pl.BlockSpec((1000, H1), lambda i: (i, 0)),
                  pl.BlockSpec((H1, 128), lambda i: (0, 0))],
        out_specs=pl.BlockSpec((1000, 128), lambda i: (i, 0)),
        out_shape=jax.ShapeDtypeStruct((N, 128), jnp.float32),
    )(p0, p1, w)


def _decode(q0, q1):
    BU = 1000
    nu = N_USERS // BU

    def body(u0, u1, i0, i1, o_ref):
        zu = u0[...] + u1[...]
        zi = i0[...] + i1[...]
        acc = lax.dot_general(zu, zi, (((1,), (1,)), ((), ())),
                              preferred_element_type=jnp.float32)
        o_ref[...] = 0.5 * jnp.tanh(acc * 0.5) + 0.5

    return pl.pallas_call(
        body,
        grid=(nu,),
        in_specs=[pl.BlockSpec((BU, 128), lambda i: (i, 0)),
                  pl.BlockSpec((BU, 128), lambda i: (i, 0)),
                  pl.BlockSpec((N_USERS, 128), lambda i: (1, 0)),
                  pl.BlockSpec((N_USERS, 128), lambda i: (1, 0))],
        out_specs=pl.BlockSpec((BU, N_USERS), lambda i: (i, 0)),
        out_shape=jax.ShapeDtypeStruct((N_USERS, N_USERS), jnp.float32),
    )(q0, q1, q0, q1)


def kernel(emb_weight, W1, W2, adj_vals, edge_index):
    rows = edge_index[0]
    cols = edge_index[1]
    vals = adj_vals
    w2p = jnp.concatenate(
        [W2, jnp.zeros((H1, 128 - H2), jnp.float32)], axis=1)
    x1 = _mm1(emb_weight, W1)
    p = _spmm128(x1, rows, cols, vals)
    x2 = _mm2(p[0], p[1], w2p)
    q = _spmm128h(x2, rows, cols, vals)
    return _decode(q[0], q[1])


# speedup vs baseline: 1.0752x; 1.0098x over previous
import functools

import jax
import jax.numpy as jnp
from jax import lax
from jax.experimental import pallas as pl
from jax.experimental.pallas import tpu as pltpu
from jax.experimental.pallas import tpu_sc as plsc

N = 10000
E = 160000
N_USERS = 5000
IN_DIM, H1, H2 = 256, 128, 64

NC, NS = 2, 16
EPC = E // NC
EPT = EPC // NS
B = 96
NB = EPT // B
TB = EPT - NB * B
ZR = 104
RPT = 624


def _make_spmm(D, d_scale):
    mesh = plsc.VectorSubcoreMesh(core_axis_name="c", subcore_axis_name="s")

    @functools.partial(
        pl.kernel,
        out_type=jax.ShapeDtypeStruct((NC, N, D), jnp.float32),
        mesh=mesh,
        scratch_types=[
            pltpu.VMEM((EPT,), jnp.int32),
            pltpu.VMEM((EPT + 16,), jnp.float32),
            pltpu.VMEM((B,), jnp.int32),
            pltpu.VMEM((B,), jnp.int32),
            pltpu.VMEM((TB,), jnp.int32),
            pltpu.VMEM((B, D), jnp.float32),
            pltpu.VMEM((B, D), jnp.float32),
            pltpu.VMEM((TB, D), jnp.float32),
            pltpu.VMEM((ZR, D), jnp.float32),
            pltpu.VMEM_SHARED((N, D), jnp.float32),
            pltpu.SemaphoreType.DMA,
            pltpu.SemaphoreType.DMA,
            pltpu.SemaphoreType.DMA,
        ],
    )
    def spmm(x_hbm, rows_hbm, cols_hbm, vals_hbm, out_hbm,
             cols_v, vals_v, rows0_v, rows1_v, rowst_v,
             msgs0_v, msgs1_v, msgst_v, zeros_v, acc_sh, sem0, sem1, sem2):
        c = lax.axis_index("c")
        s = lax.axis_index("s")
        wid = c * NS + s
        row0 = s * RPT
        ebase = wid * EPT

        bufs = ((rows0_v, msgs0_v, sem0), (rows1_v, msgs1_v, sem1))

        pltpu.async_copy(cols_hbm.at[pl.ds(ebase, EPT)], cols_v, sem2)
        pltpu.async_copy(vals_hbm.at[pl.ds(ebase, EPT)],
                         vals_v.at[pl.ds(0, EPT)], sem2)

        zv = jnp.zeros((16,), jnp.float32)

        def zfill(i, carry):
            for j in range(D // 16):
                zeros_v[i, pl.ds(16 * j, 16)] = zv
            return carry

        lax.fori_loop(0, ZR, zfill, 0)

        def zcopy(k, carry):
            pltpu.sync_copy(zeros_v, acc_sh.at[pl.ds(row0 + k * ZR, ZR)])
            return carry

        lax.fori_loop(0, RPT // ZR, zcopy, 0)

        @pl.when(s == NS - 1)
        def _():
            pltpu.sync_copy(zeros_v.at[pl.ds(0, N - NS * RPT)],
                            acc_sh.at[pl.ds(NS * RPT, N - NS * RPT)])

        pltpu.make_async_copy(cols_hbm.at[pl.ds(ebase, EPT)],
                              cols_v, sem2).wait()
        pltpu.make_async_copy(vals_hbm.at[pl.ds(ebase, EPT)],
                              vals_v.at[pl.ds(0, EPT)], sem2).wait()
        plsc.subcore_barrier()

        def scale(msgs_b, vbase, nrows):
            @plsc.parallel_loop(0, nrows, step=1, unroll=8)
            def srow(r):
                v = vals_v[pl.ds(vbase + r, 16)][0]
                for j in range(d_scale // 16):
                    sl = pl.ds(16 * j, 16)
                    msgs_b[r, sl] = msgs_b[r, sl] * v

        def issue(i, buf):
            rows_b, msgs_b, sem = bufs[buf]
            pltpu.async_copy(rows_hbm.at[pl.ds(ebase + i * B, B)],
                             rows_b, sem)
            pltpu.async_copy(x_hbm.at[cols_v.at[pl.ds(i * B, B)]],
                             msgs_b, sem)

        def consume(i, buf):
            rows_b, msgs_b, sem = bufs[buf]
            pltpu.make_async_copy(rows_hbm.at[pl.ds(ebase + i * B, B)],
                                  rows_b, sem).wait()
            pltpu.make_async_copy(x_hbm.at[cols_v.at[pl.ds(i * B, B)]],
                                  msgs_b, sem).wait()
            scale(msgs_b, i * B, B)
            pltpu.sync_copy(msgs_b, acc_sh.at[rows_b], add=True)

        def issue_tail():
            pltpu.async_copy(rows_hbm.at[pl.ds(ebase + NB * B, TB)],
                             rowst_v, sem2)
            pltpu.async_copy(x_hbm.at[cols_v.at[pl.ds(NB * B, TB)]],
                             msgst_v, sem2)

        def consume_tail():
            pltpu.make_async_copy(rows_hbm.at[pl.ds(ebase + NB * B, TB)],
                                  rowst_v, sem2).wait()
            pltpu.make_async_copy(x_hbm.at[cols_v.at[pl.ds(NB * B, TB)]],
                                  msgst_v, sem2).wait()
            scale(msgst_v, NB * B, TB)
            pltpu.sync_copy(msgst_v, acc_sh.at[rowst_v], add=True)

        issue(0, 0)

        def pair(k, carry):
            i0 = 2 * k
            issue(i0 + 1, 1)
            consume(i0, 0)
            issue(i0 + 2, 0)
            consume(i0 + 1, 1)
            return carry

        if NB % 2 == 1:
            lax.fori_loop(0, (NB - 1) // 2, pair, 0)
            issue_tail()
            consume(NB - 1, 0)
            consume_tail()
        else:
            lax.fori_loop(0, (NB - 2) // 2, pair, 0)
            issue(NB - 1, 1)
            issue_tail()
            consume(NB - 2, 0)
            consume(NB - 1, 1)
            consume_tail()

        plsc.subcore_barrier()
        pltpu.sync_copy(acc_sh.at[pl.ds(row0, RPT)],
                        out_hbm.at[c, pl.ds(row0, RPT)])

        @pl.when(s == NS - 1)
        def _():
            pltpu.sync_copy(acc_sh.at[pl.ds(NS * RPT, N - NS * RPT)],
                            out_hbm.at[c, pl.ds(NS * RPT, N - NS * RPT)])

    return spmm


_spmm128 = _make_spmm(128, 128)
_spmm128h = _make_spmm(128, H2)


def _mm1(x, w):
    def body(x_ref, w_ref, o_ref):
        o_ref[...] = jnp.dot(x_ref[...], w_ref[...],
                             preferred_element_type=jnp.float32)

    return pl.pallas_call(
        body,
        grid=(10,),
        in_specs=[pl.BlockSpec((1000, IN_DIM), lambda i: (i, 0)),
                  pl.BlockSpec((IN_DIM, H1), lambda i: (0, 0))],
        out_specs=pl.BlockSpec((1000, H1), lambda i: (i, 0)),
        out_shape=jax.ShapeDtypeStruct((N, H1), jnp.float32),
    )(x, w)


def _mm2(p0, p1, w):
    def body(p0_ref, p1_ref, w_ref, o_ref):
        h = jnp.maximum(p0_ref[...] + p1_ref[...], 0.0)
        o_ref[...] = jnp.dot(h, w_ref[...], preferred_element_type=jnp.float32)

    return pl.pallas_call(
        body,
        grid=(10,),
        in_specs=[pl.BlockSpec((1000, H1), lambda i: (i, 0)),
                  pl.BlockSpec((1000, H1), lambda i: (i, 0)),
                  pl.BlockSpec((H1, 128), lambda i: (0, 0))],
        out_specs=pl.BlockSpec((1000, 128), lambda i: (i, 0)),
        out_shape=jax.ShapeDtypeStruct((N, 128), jnp.float32),
    )(p0, p1, w)


def _decode(q0, q1):
    BU = 1000
    nu = N_USERS // BU

    def body(u0, u1, i0, i1, o_ref):
        zu = u0[...] + u1[...]
        zi = i0[...] + i1[...]
        acc = lax.dot_general(zu, zi, (((1,), (1,)), ((), ())),
                              preferred_element_type=jnp.float32)
        o_ref[...] = 0.5 * jnp.tanh(acc * 0.5) + 0.5

    return pl.pallas_call(
        body,
        grid=(nu,),
        in_specs=[pl.BlockSpec((BU, 128), lambda i: (i, 0)),
                  pl.BlockSpec((BU, 128), lambda i: (i, 0)),
                  pl.BlockSpec((N_USERS, 128), lambda i: (1, 0)),
                  pl.BlockSpec((N_USERS, 128), lambda i: (1, 0))],
        out_specs=pl.BlockSpec((BU, N_USERS), lambda i: (i, 0)),
        out_shape=jax.ShapeDtypeStruct((N_USERS, N_USERS), jnp.float32),
    )(q0, q1, q0, q1)


def kernel(emb_weight, W1, W2, adj_vals, edge_index):
    rows = edge_index[0]
    cols = edge_index[1]
    vals = adj_vals
    w2p = jnp.concatenate(
        [W2, jnp.zeros((H1, 128 - H2), jnp.float32)], axis=1)
    x1 = _mm1(emb_weight, W1)
    p = _spmm128(x1, rows, cols, vals)
    x2 = _mm2(p[0], p[1], w2p)
    q = _spmm128h(x2, rows, cols, vals)
    return _decode(q[0], q[1])
